# baseline (speedup 1.0000x reference)
import functools

import jax
import jax.numpy as jnp
from jax import lax
from jax.experimental import pallas as pl
from jax.experimental.pallas import tpu as pltpu
from jax.experimental.pallas import tpu_sc as plsc

N_USER = 10000
N_SRC = 10000
E = 320000
D = 128

_LANES = 16
_NTILES = 16
_ROWS_PER_IDX = 64
_GROUP = 8
_GBUF = 4
_GROUPS_PER_TILE = 40
_MAIN_GROUPS = _GROUPS_PER_TILE - 1
_NROWS = E // _ROWS_PER_IDX
_TAIL_ROWS = _NTILES * _GROUP
_E_TAIL = _TAIL_ROWS * _ROWS_PER_IDX
_E_MAIN = _NTILES * _MAIN_GROUPS * _GROUP * _ROWS_PER_IDX
_N_ACC = N_USER + 16
_ROWS_OUT = 624
_ROWS_TAIL = N_USER - _ROWS_OUT * _NTILES


def _sc_segment_sums(x_image, x_text, src_img, dst_img, src_txt, dst_txt,
                     ts_img, td_img, ts_txt, td_txt, zrows, zcnt):
  mesh = plsc.VectorSubcoreMesh(core_axis_name="c", subcore_axis_name="s")

  @functools.partial(
      pl.kernel,
      out_type=(
          jax.ShapeDtypeStruct((2, N_USER, D), jnp.float32),
          jax.ShapeDtypeStruct((2, _N_ACC), jnp.float32),
      ),
      mesh=mesh,
      scratch_types=[
          pltpu.VMEM((_GROUP, _ROWS_PER_IDX), jnp.int32),
          pltpu.VMEM((_GROUP, _ROWS_PER_IDX), jnp.int32),
          pltpu.VMEM((_GROUP, _ROWS_PER_IDX), jnp.int32),
          pltpu.VMEM((_GROUP, _ROWS_PER_IDX), jnp.int32),
          pltpu.VMEM((_GBUF, _ROWS_PER_IDX, D), jnp.float32),
          pltpu.VMEM((_ROWS_PER_IDX,), jnp.float32),
          pltpu.VMEM_SHARED((_N_ACC, D), jnp.float32),
          pltpu.VMEM_SHARED((_N_ACC,), jnp.float32),
          pltpu.SemaphoreType.DMA,
          pltpu.SemaphoreType.DMA,
          pltpu.SemaphoreType.DMA,
          pltpu.SemaphoreType.DMA,
      ],
  )
  def seg_kernel(x_img_hbm, x_txt_hbm, s_img_hbm, d_img_hbm, s_txt_hbm,
                 d_txt_hbm, ts_img_hbm, td_img_hbm, ts_txt_hbm, td_txt_hbm,
                 zrows_hbm, zcnt_hbm, out_s_hbm, out_c_hbm,
                 sidx0, didx0, sidx1, didx1, rows, ones, acc, cnt,
                 gsem, isem, csem, ssem):
    cid = lax.axis_index("c")
    sid = lax.axis_index("s")

    pltpu.sync_copy(zrows_hbm.at[pl.ds(0, _ROWS_OUT), :],
                    acc.at[pl.ds(sid * _ROWS_OUT, _ROWS_OUT), :])
    @pl.when(sid == 0)
    def _():
      pltpu.sync_copy(zcnt_hbm, cnt)
      pltpu.sync_copy(zrows_hbm.at[pl.ds(0, _ROWS_TAIL), :],
                      acc.at[pl.ds(_ROWS_OUT * _NTILES, _ROWS_TAIL), :])
    for i in range(_ROWS_PER_IDX // _LANES):
      ones[pl.ds(i * _LANES, _LANES)] = jnp.ones((_LANES,), jnp.float32)
    plsc.subcore_barrier()

    def main_loop(x_tbl, src2, dst2, tsrc, tdst):
      def fire_idx(k, sP, dP):
        rb = (sid + k * _NTILES) * _GROUP
        pltpu.async_copy(src2.at[pl.ds(rb, _GROUP), :], sP, isem)
        pltpu.async_copy(dst2.at[pl.ds(rb, _GROUP), :], dP, isem)

      def wait_idx(sP, dP):
        pltpu.make_async_copy(src2.at[pl.ds(0, _GROUP), :], sP, isem).wait()
        pltpu.make_async_copy(dst2.at[pl.ds(0, _GROUP), :], dP, isem).wait()

      def fire_gather(sP, r, slot):
        pltpu.async_copy(x_tbl.at[sP.at[r]], rows.at[slot], gsem)

      def wait_gathers():
        for slot in range(_GBUF):
          pltpu.make_async_copy(x_tbl.at[sidx0.at[0]], rows.at[slot],
                                gsem).wait()

      def fire_cnt(dP, r):
        return pltpu.async_copy(ones, cnt.at[dP.at[r]], csem, add=True)

      def half(dP_sc, r0_sc, sP_g, r0_g):
        wait_gathers()
        hs = [pltpu.async_copy(rows.at[j], acc.at[dP_sc.at[r0_sc + j]],
                               ssem, add=True) for j in range(_GBUF)]
        for h in hs:
          h.wait()
        for j in range(_GBUF):
          if sP_g is not None:
            fire_gather(sP_g, r0_g + j, j)
        return [fire_cnt(dP_sc, r0_sc + j) for j in range(_GBUF)]

      fire_idx(0, sidx0, didx0)
      wait_idx(sidx0, didx0)
      for j in range(_GBUF):
        fire_gather(sidx0, j, j)

      def pair_body(i, _):
        kA = 2 * i
        fire_idx(kA + 1, sidx1, didx1)
        ch = half(didx0, 0, sidx0, _GBUF)
        wait_idx(sidx1, didx1)
        ch += half(didx0, _GBUF, sidx1, 0)
        for h in ch:
          h.wait()
        fire_idx(kA + 2, sidx0, didx0)
        ch = half(didx1, 0, sidx1, _GBUF)
        wait_idx(sidx0, didx0)
        ch += half(didx1, _GBUF, sidx0, 0)
        for h in ch:
          h.wait()
        return 0

      lax.fori_loop(0, (_GROUPS_PER_TILE - 2) // 2, pair_body, 0)

      rbt = sid * _GROUP
      pltpu.async_copy(tsrc.at[pl.ds(rbt, _GROUP), :], sidx1, isem)
      pltpu.async_copy(tdst.at[pl.ds(rbt, _GROUP), :], didx1, isem)
      ch = half(didx0, 0, sidx0, _GBUF)
      wait_idx(sidx1, didx1)
      ch += half(didx0, _GBUF, sidx1, 0)
      for h in ch:
        h.wait()
      ch = half(didx1, 0, sidx1, _GBUF)
      ch += half(didx1, _GBUF, None, 0)
      for h in ch:
        h.wait()

    @pl.when(cid == 0)
    def _():
      main_loop(x_img_hbm, s_img_hbm, d_img_hbm, ts_img_hbm, td_img_hbm)
    @pl.when(cid == 1)
    def _():
      main_loop(x_txt_hbm, s_txt_hbm, d_txt_hbm, ts_txt_hbm, td_txt_hbm)

    plsc.subcore_barrier()

    def writeout(rel):
      pltpu.sync_copy(acc.at[pl.ds(sid * _ROWS_OUT, _ROWS_OUT), :],
                      out_s_hbm.at[rel, pl.ds(sid * _ROWS_OUT, _ROWS_OUT), :])
      @pl.when(sid == 0)
      def _():
        pltpu.sync_copy(cnt, out_c_hbm.at[rel])
        pltpu.sync_copy(
            acc.at[pl.ds(_ROWS_OUT * _NTILES, _ROWS_TAIL), :],
            out_s_hbm.at[rel, pl.ds(_ROWS_OUT * _NTILES, _ROWS_TAIL), :])

    @pl.when(cid == 0)
    def _():
      writeout(0)
    @pl.when(cid == 1)
    def _():
      writeout(1)

  return seg_kernel(x_image, x_text, src_img, dst_img, src_txt, dst_txt,
                    ts_img, td_img, ts_txt, td_txt, zrows, zcnt)


def _tc_combine(x_user, sums, cnts3, W_user, b_user2, Wl_img, Wl_txt,
                Wr_img, Wr_txt, bl_img2, bl_txt2):
  blk = 1000
  grid = (N_USER // blk,)

  def dotT(a, b):
    return lax.dot_general(a, b, (((1,), (1,)), ((), ())),
                           preferred_element_type=jnp.float32)

  def body(xu_ref, si_ref, st_ref, ci_ref, ct_ref, Wu_ref, bu_ref,
           Wli_ref, Wlt_ref, Wri_ref, Wrt_ref, bli_ref, blt_ref, out_ref):
    xu = dotT(xu_ref[...], Wu_ref[...]) + bu_ref[...]
    ci = jnp.maximum(ci_ref[0, :, :], 1.0)
    ct = jnp.maximum(ct_ref[0, :, :], 1.0)
    mi = si_ref[0] / ci
    mt = st_ref[0] / ct
    out = (dotT(mi, Wli_ref[...]) + dotT(mt, Wlt_ref[...])
           + dotT(xu, Wri_ref[...]) + dotT(xu, Wrt_ref[...])
           + bli_ref[...] + blt_ref[...])
    out_ref[...] = jnp.maximum(out, 0.0)

  full2 = pl.BlockSpec((128, 128), lambda i: (0, 0))
  bias2 = pl.BlockSpec((1, 128), lambda i: (0, 0))
  return pl.pallas_call(
      body,
      grid=grid,
      in_specs=[
          pl.BlockSpec((blk, D), lambda i: (i, 0)),
          pl.BlockSpec((1, blk, D), lambda i: (0, i, 0)),
          pl.BlockSpec((1, blk, D), lambda i: (1, i, 0)),
          pl.BlockSpec((1, blk, 1), lambda i: (0, i, 0)),
          pl.BlockSpec((1, blk, 1), lambda i: (1, i, 0)),
          full2, bias2, full2, full2, full2, full2, bias2, bias2,
      ],
      out_specs=pl.BlockSpec((blk, D), lambda i: (i, 0)),
      out_shape=jax.ShapeDtypeStruct((N_USER, D), jnp.float32),
  )(x_user, sums, sums, cnts3, cnts3, W_user, b_user2, Wl_img, Wl_txt,
    Wr_img, Wr_txt, bl_img2, bl_txt2)


def kernel(x_user, x_image, x_text, edge_index_image_user,
           edge_index_text_user, W_user, b_user, Wl_img, bl_img, Wr_img,
           Wl_txt, bl_txt, Wr_txt):
  def tail(e, pad):
    return (jnp.concatenate([e[_E_MAIN:], pad])
            .reshape(_TAIL_ROWS, _ROWS_PER_IDX))

  n_pad = _E_TAIL - (E - _E_MAIN)
  src_pad = jnp.arange(n_pad, dtype=jnp.int32) % N_SRC
  dst_pad = N_USER + (jnp.arange(n_pad, dtype=jnp.int32) % 16)
  src_img = edge_index_image_user[0].reshape(_NROWS, _ROWS_PER_IDX)
  dst_img = edge_index_image_user[1].reshape(_NROWS, _ROWS_PER_IDX)
  src_txt = edge_index_text_user[0].reshape(_NROWS, _ROWS_PER_IDX)
  dst_txt = edge_index_text_user[1].reshape(_NROWS, _ROWS_PER_IDX)
  ts_img = tail(edge_index_image_user[0], src_pad)
  td_img = tail(edge_index_image_user[1], dst_pad)
  ts_txt = tail(edge_index_text_user[0], src_pad)
  td_txt = tail(edge_index_text_user[1], dst_pad)
  zrows = jnp.zeros((_ROWS_OUT, D), jnp.float32)
  zcnt = jnp.zeros((_N_ACC,), jnp.float32)

  sums, cnts = _sc_segment_sums(x_image, x_text, src_img, dst_img,
                                src_txt, dst_txt, ts_img, td_img,
                                ts_txt, td_txt, zrows, zcnt)

  return _tc_combine(
      x_user, sums, cnts[:, :N_USER].reshape(2, N_USER, 1), W_user,
      b_user.reshape(1, D), Wl_img, Wl_txt, Wr_img, Wr_txt,
      bl_img.reshape(1, D), bl_txt.reshape(1, D))


# speedup vs baseline: 1.1770x; 1.1770x over previous
import functools

import jax
import jax.numpy as jnp
from jax import lax
from jax.experimental import pallas as pl
from jax.experimental.pallas import tpu as pltpu
from jax.experimental.pallas import tpu_sc as plsc

N_USER = 10000
N_SRC = 10000
E = 320000
D = 128

_LANES = 16
_NTILES = 16
_ROWS_PER_IDX = 64
_GROUP = 8
_GBUF = 4
_GROUPS_PER_TILE = 40
_MAIN_GROUPS = _GROUPS_PER_TILE - 1
_NROWS = E // _ROWS_PER_IDX
_TAIL_ROWS = _NTILES * _GROUP
_E_TAIL = _TAIL_ROWS * _ROWS_PER_IDX
_E_MAIN = _NTILES * _MAIN_GROUPS * _GROUP * _ROWS_PER_IDX
_N_ACC = N_USER + 16
_ROWS_OUT = 624
_ROWS_TAIL = N_USER - _ROWS_OUT * _NTILES


def _sc_segment_sums(x_image, x_text, src_img, dst_img, src_txt, dst_txt,
                     ts_img, td_img, ts_txt, td_txt, zrows, zcnt):
  mesh = plsc.VectorSubcoreMesh(core_axis_name="c", subcore_axis_name="s")

  @functools.partial(
      pl.kernel,
      out_type=(
          jax.ShapeDtypeStruct((2, N_USER, D), jnp.float32),
          jax.ShapeDtypeStruct((2, _N_ACC), jnp.float32),
      ),
      mesh=mesh,
      scratch_types=[
          pltpu.VMEM((_GROUP, _ROWS_PER_IDX), jnp.int32),
          pltpu.VMEM((_GROUP, _ROWS_PER_IDX), jnp.int32),
          pltpu.VMEM((_GROUP, _ROWS_PER_IDX), jnp.int32),
          pltpu.VMEM((_GROUP, _ROWS_PER_IDX), jnp.int32),
          pltpu.VMEM((_GBUF, _ROWS_PER_IDX, D), jnp.float32),
          pltpu.VMEM((_ROWS_PER_IDX,), jnp.float32),
          pltpu.VMEM_SHARED((_N_ACC, D), jnp.float32),
          pltpu.VMEM_SHARED((_N_ACC,), jnp.float32),
          pltpu.SemaphoreType.DMA,
          pltpu.SemaphoreType.DMA,
          pltpu.SemaphoreType.DMA,
          pltpu.SemaphoreType.DMA,
      ],
  )
  def seg_kernel(x_img_hbm, x_txt_hbm, s_img_hbm, d_img_hbm, s_txt_hbm,
                 d_txt_hbm, ts_img_hbm, td_img_hbm, ts_txt_hbm, td_txt_hbm,
                 zrows_hbm, zcnt_hbm, out_s_hbm, out_c_hbm,
                 sidx0, didx0, sidx1, didx1, rows, ones, acc, cnt,
                 gsem, isem, csem, ssem):
    cid = lax.axis_index("c")
    sid = lax.axis_index("s")

    pltpu.sync_copy(zrows_hbm.at[pl.ds(0, _ROWS_OUT), :],
                    acc.at[pl.ds(sid * _ROWS_OUT, _ROWS_OUT), :])
    @pl.when(sid == 0)
    def _():
      pltpu.sync_copy(zcnt_hbm, cnt)
      pltpu.sync_copy(zrows_hbm.at[pl.ds(0, _ROWS_TAIL), :],
                      acc.at[pl.ds(_ROWS_OUT * _NTILES, _ROWS_TAIL), :])
    for i in range(_ROWS_PER_IDX // _LANES):
      ones[pl.ds(i * _LANES, _LANES)] = jnp.ones((_LANES,), jnp.float32)
    plsc.subcore_barrier()

    def main_loop(x_tbl, src2, dst2, tsrc, tdst):
      def fire_idx(k, sP, dP):
        rb = (sid + k * _NTILES) * _GROUP
        pltpu.async_copy(src2.at[pl.ds(rb, _GROUP), :], sP, isem)
        pltpu.async_copy(dst2.at[pl.ds(rb, _GROUP), :], dP, isem)

      def wait_idx(sP, dP):
        pltpu.make_async_copy(src2.at[pl.ds(0, _GROUP), :], sP, isem).wait()
        pltpu.make_async_copy(dst2.at[pl.ds(0, _GROUP), :], dP, isem).wait()

      def fire_gather(sP, r, slot):
        pltpu.async_copy(x_tbl.at[sP.at[r]], rows.at[slot], gsem)

      def wait_gathers():
        for slot in range(_GBUF):
          pltpu.make_async_copy(x_tbl.at[sidx0.at[0]], rows.at[slot],
                                gsem).wait()

      def fire_cnt(dP, r):
        return pltpu.async_copy(ones, cnt.at[dP.at[r]], csem, add=True)

      def half(dP_sc, r0_sc, sP_g, r0_g):
        wait_gathers()
        for j in range(_GBUF):
          pltpu.sync_copy(rows.at[j], acc.at[dP_sc.at[r0_sc + j]], add=True)
          if sP_g is not None:
            fire_gather(sP_g, r0_g + j, j)
        return [fire_cnt(dP_sc, r0_sc + j) for j in range(_GBUF)]

      fire_idx(0, sidx0, didx0)
      wait_idx(sidx0, didx0)
      for j in range(_GBUF):
        fire_gather(sidx0, j, j)

      def pair_body(i, _):
        kA = 2 * i
        fire_idx(kA + 1, sidx1, didx1)
        ch = half(didx0, 0, sidx0, _GBUF)
        wait_idx(sidx1, didx1)
        ch += half(didx0, _GBUF, sidx1, 0)
        for h in ch:
          h.wait()
        fire_idx(kA + 2, sidx0, didx0)
        ch = half(didx1, 0, sidx1, _GBUF)
        wait_idx(sidx0, didx0)
        ch += half(didx1, _GBUF, sidx0, 0)
        for h in ch:
          h.wait()
        return 0

      lax.fori_loop(0, (_GROUPS_PER_TILE - 2) // 2, pair_body, 0)

      rbt = sid * _GROUP
      pltpu.async_copy(tsrc.at[pl.ds(rbt, _GROUP), :], sidx1, isem)
      pltpu.async_copy(tdst.at[pl.ds(rbt, _GROUP), :], didx1, isem)
      ch = half(didx0, 0, sidx0, _GBUF)
      wait_idx(sidx1, didx1)
      ch += half(didx0, _GBUF, sidx1, 0)
      for h in ch:
        h.wait()
      ch = half(didx1, 0, sidx1, _GBUF)
      ch += half(didx1, _GBUF, None, 0)
      for h in ch:
        h.wait()

    @pl.when(cid == 0)
    def _():
      main_loop(x_img_hbm, s_img_hbm, d_img_hbm, ts_img_hbm, td_img_hbm)
    @pl.when(cid == 1)
    def _():
      main_loop(x_txt_hbm, s_txt_hbm, d_txt_hbm, ts_txt_hbm, td_txt_hbm)

    plsc.subcore_barrier()

    def writeout(rel):
      pltpu.sync_copy(acc.at[pl.ds(sid * _ROWS_OUT, _ROWS_OUT), :],
                      out_s_hbm.at[rel, pl.ds(sid * _ROWS_OUT, _ROWS_OUT), :])
      @pl.when(sid == 0)
      def _():
        pltpu.sync_copy(cnt, out_c_hbm.at[rel])
        pltpu.sync_copy(
            acc.at[pl.ds(_ROWS_OUT * _NTILES, _ROWS_TAIL), :],
            out_s_hbm.at[rel, pl.ds(_ROWS_OUT * _NTILES, _ROWS_TAIL), :])

    @pl.when(cid == 0)
    def _():
      writeout(0)
    @pl.when(cid == 1)
    def _():
      writeout(1)

  return seg_kernel(x_image, x_text, src_img, dst_img, src_txt, dst_txt,
                    ts_img, td_img, ts_txt, td_txt, zrows, zcnt)


def _tc_combine(x_user, sums, cnts3, W_user, b_user2, Wl_img, Wl_txt,
                Wr_img, Wr_txt, bl_img2, bl_txt2):
  blk = 1000
  grid = (N_USER // blk,)

  def dotT(a, b):
    return lax.dot_general(a, b, (((1,), (1,)), ((), ())),
                           preferred_element_type=jnp.float32)

  def body(xu_ref, si_ref, st_ref, ci_ref, ct_ref, Wu_ref, bu_ref,
           Wli_ref, Wlt_ref, Wri_ref, Wrt_ref, bli_ref, blt_ref, out_ref):
    xu = dotT(xu_ref[...], Wu_ref[...]) + bu_ref[...]
    ci = jnp.maximum(ci_ref[0, :, :], 1.0)
    ct = jnp.maximum(ct_ref[0, :, :], 1.0)
    mi = si_ref[0] / ci
    mt = st_ref[0] / ct
    out = (dotT(mi, Wli_ref[...]) + dotT(mt, Wlt_ref[...])
           + dotT(xu, Wri_ref[...]) + dotT(xu, Wrt_ref[...])
           + bli_ref[...] + blt_ref[...])
    out_ref[...] = jnp.maximum(out, 0.0)

  full2 = pl.BlockSpec((128, 128), lambda i: (0, 0))
  bias2 = pl.BlockSpec((1, 128), lambda i: (0, 0))
  return pl.pallas_call(
      body,
      grid=grid,
      in_specs=[
          pl.BlockSpec((blk, D), lambda i: (i, 0)),
          pl.BlockSpec((1, blk, D), lambda i: (0, i, 0)),
          pl.BlockSpec((1, blk, D), lambda i: (1, i, 0)),
          pl.BlockSpec((1, blk, 1), lambda i: (0, i, 0)),
          pl.BlockSpec((1, blk, 1), lambda i: (1, i, 0)),
          full2, bias2, full2, full2, full2, full2, bias2, bias2,
      ],
      out_specs=pl.BlockSpec((blk, D), lambda i: (i, 0)),
      out_shape=jax.ShapeDtypeStruct((N_USER, D), jnp.float32),
  )(x_user, sums, sums, cnts3, cnts3, W_user, b_user2, Wl_img, Wl_txt,
    Wr_img, Wr_txt, bl_img2, bl_txt2)


def kernel(x_user, x_image, x_text, edge_index_image_user,
           edge_index_text_user, W_user, b_user, Wl_img, bl_img, Wr_img,
           Wl_txt, bl_txt, Wr_txt):
  def tail(e, pad):
    return (jnp.concatenate([e[_E_MAIN:], pad])
            .reshape(_TAIL_ROWS, _ROWS_PER_IDX))

  n_pad = _E_TAIL - (E - _E_MAIN)
  src_pad = jnp.arange(n_pad, dtype=jnp.int32) % N_SRC
  dst_pad = N_USER + (jnp.arange(n_pad, dtype=jnp.int32) % 16)
  src_img = edge_index_image_user[0].reshape(_NROWS, _ROWS_PER_IDX)
  dst_img = edge_index_image_user[1].reshape(_NROWS, _ROWS_PER_IDX)
  src_txt = edge_index_text_user[0].reshape(_NROWS, _ROWS_PER_IDX)
  dst_txt = edge_index_text_user[1].reshape(_NROWS, _ROWS_PER_IDX)
  ts_img = tail(edge_index_image_user[0], src_pad)
  td_img = tail(edge_index_image_user[1], dst_pad)
  ts_txt = tail(edge_index_text_user[0], src_pad)
  td_txt = tail(edge_index_text_user[1], dst_pad)
  zrows = jnp.zeros((_ROWS_OUT, D), jnp.float32)
  zcnt = jnp.zeros((_N_ACC,), jnp.float32)

  sums, cnts = _sc_segment_sums(x_image, x_text, src_img, dst_img,
                                src_txt, dst_txt, ts_img, td_img,
                                ts_txt, td_txt, zrows, zcnt)

  return _tc_combine(
      x_user, sums, cnts[:, :N_USER].reshape(2, N_USER, 1), W_user,
      b_user.reshape(1, D), Wl_img, Wl_txt, Wr_img, Wr_txt,
      bl_img.reshape(1, D), bl_txt.reshape(1, D))


# speedup vs baseline: 1.4634x; 1.2432x over previous
import functools

import jax
import jax.numpy as jnp
from jax import lax
from jax.experimental import pallas as pl
from jax.experimental.pallas import tpu as pltpu
from jax.experimental.pallas import tpu_sc as plsc

N_USER = 10000
N_SRC = 10000
E = 320000
D = 128

_LANES = 16
_NTILES = 16
_ROWS_PER_IDX = 64
_GROUP = 8
_GBUF = 4
_GROUPS_PER_TILE = 40
_NGROUPS = _NTILES * _GROUPS_PER_TILE
_NROWS = _NGROUPS * _GROUP
_E_PAD = _NROWS * _ROWS_PER_IDX
_N_ACC = N_USER + 16
_ROWS_OUT = 624
_ROWS_TAIL = N_USER - _ROWS_OUT * _NTILES


def _sc_segment_sums(x_image, x_text, src_img, dst_img, src_txt, dst_txt,
                     zrows, zcnt):
  mesh = plsc.VectorSubcoreMesh(core_axis_name="c", subcore_axis_name="s")

  @functools.partial(
      pl.kernel,
      out_type=(
          jax.ShapeDtypeStruct((2, N_USER, D), jnp.float32),
          jax.ShapeDtypeStruct((2, _N_ACC), jnp.float32),
      ),
      mesh=mesh,
      scratch_types=[
          pltpu.VMEM((_GROUP, _ROWS_PER_IDX), jnp.int32),
          pltpu.VMEM((_GROUP, _ROWS_PER_IDX), jnp.int32),
          pltpu.VMEM((_GROUP, _ROWS_PER_IDX), jnp.int32),
          pltpu.VMEM((_GROUP, _ROWS_PER_IDX), jnp.int32),
          pltpu.VMEM((_GBUF, _ROWS_PER_IDX, D), jnp.float32),
          pltpu.VMEM((_ROWS_PER_IDX,), jnp.float32),
          pltpu.VMEM_SHARED((_N_ACC, D), jnp.float32),
          pltpu.VMEM_SHARED((_N_ACC,), jnp.float32),
          pltpu.SemaphoreType.DMA,
          pltpu.SemaphoreType.DMA,
          pltpu.SemaphoreType.DMA,
          pltpu.SemaphoreType.DMA,
          pltpu.SemaphoreType.DMA,
          pltpu.SemaphoreType.DMA,
      ],
  )
  def seg_kernel(x_img_hbm, x_txt_hbm, s_img_hbm, d_img_hbm, s_txt_hbm,
                 d_txt_hbm, zrows_hbm, zcnt_hbm, out_s_hbm, out_c_hbm,
                 sidx0, didx0, sidx1, didx1, rows, ones, acc, cnt,
                 gsem0, gsem1, gsem2, gsem3, isem, csem):
    cid = lax.axis_index("c")
    sid = lax.axis_index("s")

    pltpu.sync_copy(zrows_hbm.at[pl.ds(0, _ROWS_OUT), :],
                    acc.at[pl.ds(sid * _ROWS_OUT, _ROWS_OUT), :])
    @pl.when(sid == 0)
    def _():
      pltpu.sync_copy(zcnt_hbm, cnt)
      pltpu.sync_copy(zrows_hbm.at[pl.ds(0, _ROWS_TAIL), :],
                      acc.at[pl.ds(_ROWS_OUT * _NTILES, _ROWS_TAIL), :])
    for i in range(_ROWS_PER_IDX // _LANES):
      ones[pl.ds(i * _LANES, _LANES)] = jnp.ones((_LANES,), jnp.float32)
    plsc.subcore_barrier()

    def main_loop(x_tbl, src2, dst2):
      def fire_idx(k, sP, dP):
        rb = (sid + k * _NTILES) * _GROUP
        pltpu.async_copy(src2.at[pl.ds(rb, _GROUP), :], sP, isem)
        pltpu.async_copy(dst2.at[pl.ds(rb, _GROUP), :], dP, isem)

      def wait_idx(sP, dP):
        pltpu.make_async_copy(src2.at[pl.ds(0, _GROUP), :], sP, isem).wait()
        pltpu.make_async_copy(dst2.at[pl.ds(0, _GROUP), :], dP, isem).wait()

      gsems = [gsem0, gsem1, gsem2, gsem3]

      def fire_gather(sP, r, slot):
        pltpu.async_copy(x_tbl.at[sP.at[r]], rows.at[slot], gsems[slot])

      def wait_gather(slot):
        pltpu.make_async_copy(x_tbl.at[sidx0.at[0]], rows.at[slot],
                              gsems[slot]).wait()

      def scatter(dP, r, slot):
        pltpu.sync_copy(rows.at[slot], acc.at[dP.at[r]], add=True)

      def fire_cnt(dP, r):
        return pltpu.async_copy(ones, cnt.at[dP.at[r]], csem, add=True)

      def half(dP_sc, r0_sc, sP_g, r0_g):
        for j in range(_GBUF):
          wait_gather(j)
          scatter(dP_sc, r0_sc + j, j)
          if sP_g is not None:
            fire_gather(sP_g, r0_g + j, j)
        return [fire_cnt(dP_sc, r0_sc + j) for j in range(_GBUF)]

      fire_idx(0, sidx0, didx0)
      wait_idx(sidx0, didx0)
      for j in range(_GBUF):
        fire_gather(sidx0, j, j)

      def pair_body(i, _):
        kA = 2 * i
        fire_idx(kA + 1, sidx1, didx1)
        ch = half(didx0, 0, sidx0, _GBUF)
        wait_idx(sidx1, didx1)
        ch += half(didx0, _GBUF, sidx1, 0)
        for h in ch:
          h.wait()
        fire_idx(kA + 2, sidx0, didx0)
        ch = half(didx1, 0, sidx1, _GBUF)
        wait_idx(sidx0, didx0)
        ch += half(didx1, _GBUF, sidx0, 0)
        for h in ch:
          h.wait()
        return 0

      lax.fori_loop(0, (_GROUPS_PER_TILE - 2) // 2, pair_body, 0)

      fire_idx(_GROUPS_PER_TILE - 1, sidx1, didx1)
      ch = half(didx0, 0, sidx0, _GBUF)
      wait_idx(sidx1, didx1)
      ch += half(didx0, _GBUF, sidx1, 0)
      for h in ch:
        h.wait()
      ch = half(didx1, 0, sidx1, _GBUF)
      ch += half(didx1, _GBUF, None, 0)
      for h in ch:
        h.wait()

    @pl.when(cid == 0)
    def _():
      main_loop(x_img_hbm, s_img_hbm, d_img_hbm)
    @pl.when(cid == 1)
    def _():
      main_loop(x_txt_hbm, s_txt_hbm, d_txt_hbm)

    plsc.subcore_barrier()

    def writeout(rel):
      pltpu.sync_copy(acc.at[pl.ds(sid * _ROWS_OUT, _ROWS_OUT), :],
                      out_s_hbm.at[rel, pl.ds(sid * _ROWS_OUT, _ROWS_OUT), :])
      @pl.when(sid == 0)
      def _():
        pltpu.sync_copy(cnt, out_c_hbm.at[rel])
        pltpu.sync_copy(
            acc.at[pl.ds(_ROWS_OUT * _NTILES, _ROWS_TAIL), :],
            out_s_hbm.at[rel, pl.ds(_ROWS_OUT * _NTILES, _ROWS_TAIL), :])

    @pl.when(cid == 0)
    def _():
      writeout(0)
    @pl.when(cid == 1)
    def _():
      writeout(1)

  return seg_kernel(x_image, x_text, src_img, dst_img, src_txt, dst_txt,
                    zrows, zcnt)


def _tc_combine(x_user, sums, cnts3, W_user, b_user2, Wl_img, Wl_txt,
                Wr_img, Wr_txt, bl_img2, bl_txt2):
  blk = 1000
  grid = (N_USER // blk,)

  def dotT(a, b):
    return lax.dot_general(a, b, (((1,), (1,)), ((), ())),
                           preferred_element_type=jnp.float32)

  def body(xu_ref, si_ref, st_ref, ci_ref, ct_ref, Wu_ref, bu_ref,
           Wli_ref, Wlt_ref, Wri_ref, Wrt_ref, bli_ref, blt_ref, out_ref):
    xu = dotT(xu_ref[...], Wu_ref[...]) + bu_ref[...]
    ci = jnp.maximum(ci_ref[0, :, :], 1.0)
    ct = jnp.maximum(ct_ref[0, :, :], 1.0)
    mi = si_ref[0] / ci
    mt = st_ref[0] / ct
    out = (dotT(mi, Wli_ref[...]) + dotT(mt, Wlt_ref[...])
           + dotT(xu, Wri_ref[...]) + dotT(xu, Wrt_ref[...])
           + bli_ref[...] + blt_ref[...])
    out_ref[...] = jnp.maximum(out, 0.0)

  full2 = pl.BlockSpec((128, 128), lambda i: (0, 0))
  bias2 = pl.BlockSpec((1, 128), lambda i: (0, 0))
  return pl.pallas_call(
      body,
      grid=grid,
      in_specs=[
          pl.BlockSpec((blk, D), lambda i: (i, 0)),
          pl.BlockSpec((1, blk, D), lambda i: (0, i, 0)),
          pl.BlockSpec((1, blk, D), lambda i: (1, i, 0)),
          pl.BlockSpec((1, blk, 1), lambda i: (0, i, 0)),
          pl.BlockSpec((1, blk, 1), lambda i: (1, i, 0)),
          full2, bias2, full2, full2, full2, full2, bias2, bias2,
      ],
      out_specs=pl.BlockSpec((blk, D), lambda i: (i, 0)),
      out_shape=jax.ShapeDtypeStruct((N_USER, D), jnp.float32),
  )(x_user, sums, sums, cnts3, cnts3, W_user, b_user2, Wl_img, Wl_txt,
    Wr_img, Wr_txt, bl_img2, bl_txt2)


def kernel(x_user, x_image, x_text, edge_index_image_user,
           edge_index_text_user, W_user, b_user, Wl_img, bl_img, Wr_img,
           Wl_txt, bl_txt, Wr_txt):
  def prep(e, pad):
    return jnp.concatenate([e, pad]).reshape(_NROWS, _ROWS_PER_IDX)

  src_pad = jnp.arange(_E_PAD - E, dtype=jnp.int32) % N_SRC
  dst_pad = N_USER + (jnp.arange(_E_PAD - E, dtype=jnp.int32) % 16)
  src_img = prep(edge_index_image_user[0], src_pad)
  dst_img = prep(edge_index_image_user[1], dst_pad)
  src_txt = prep(edge_index_text_user[0], src_pad)
  dst_txt = prep(edge_index_text_user[1], dst_pad)
  zrows = jnp.zeros((_ROWS_OUT, D), jnp.float32)
  zcnt = jnp.zeros((_N_ACC,), jnp.float32)

  sums, cnts = _sc_segment_sums(x_image, x_text, src_img, dst_img,
                                src_txt, dst_txt, zrows, zcnt)

  return _tc_combine(
      x_user, sums, cnts[:, :N_USER].reshape(2, N_USER, 1), W_user,
      b_user.reshape(1, D), Wl_img, Wl_txt, Wr_img, Wr_txt,
      bl_img.reshape(1, D), bl_txt.reshape(1, D))
